# baseline (speedup 1.0000x reference)
import functools
import math

import jax
import jax.numpy as jnp
import numpy as np
from jax import lax
from jax.experimental import pallas as pl
from jax.experimental.pallas import tpu as pltpu
from jax.experimental.pallas import tpu_sc as plsc

D = 512
B = 4
S = 2048
NFLAT = B * S
SCALE = math.sqrt(D)

NC, NS, L = 2, 16, 16
NW = NC * NS
POS_PER_W = S // NW


def _positional_encoding() -> np.ndarray:
    position = np.arange(S, dtype=np.float32)[:, None]
    div_term = np.exp(
        np.arange(0, D, 2, dtype=np.float32) * (-math.log(10000.0) / D)
    )
    pe = np.zeros((S, D), dtype=np.float32)
    pe[:, 0::2] = np.sin(position * div_term)
    pe[:, 1::2] = np.cos(position * div_term)
    return pe


def _pe_packed() -> np.ndarray:
    pe = _positional_encoding().astype(jnp.bfloat16)
    inter = pe.reshape(S, D // 32, 2, 16).transpose(0, 1, 3, 2).reshape(S, D)
    return inter.view(np.uint16).view(np.int32)


_PE_PACKED = _pe_packed()


def _make_kernel():
    mesh = plsc.VectorSubcoreMesh(core_axis_name="c", subcore_axis_name="s")

    @functools.partial(
        pl.kernel,
        mesh=mesh,
        out_type=jax.ShapeDtypeStruct((NFLAT, D), jnp.float32),
        scratch_types=[
            pltpu.VMEM((B, POS_PER_W), jnp.int32),
            pltpu.VMEM((POS_PER_W, D // 2), jnp.int32),
            pltpu.VMEM((POS_PER_W, D), jnp.float32),
            pltpu.VMEM((POS_PER_W, D), jnp.float32),
            pltpu.VMEM((POS_PER_W, D), jnp.float32),
            pltpu.SemaphoreType.DMA,
            pltpu.SemaphoreType.DMA,
            pltpu.SemaphoreType.DMA,
            pltpu.SemaphoreType.DMA,
            pltpu.SemaphoreType.DMA,
            pltpu.SemaphoreType.DMA,
        ],
    )
    def emb(x_hbm, table_hbm, pe_hbm, out_hbm,
            idx_v, pack_v, pe_v, rows0, rows1,
            isem, psem, g0, g1, o0, o1):
        wid = lax.axis_index("s") * NC + lax.axis_index("c")
        pos0 = wid * POS_PER_W

        i_h = [
            pltpu.async_copy(
                x_hbm.at[b, pl.ds(pos0, POS_PER_W)], idx_v.at[b], isem)
            for b in range(B)
        ]
        p_h = pltpu.async_copy(
            pe_hbm.at[pl.ds(pos0, POS_PER_W)], pack_v, psem)
        for h in i_h:
            h.wait()

        rows = (rows0, rows1)
        gsem = (g0, g1)
        osem = (o0, o1)
        g_h = [None, None]
        o_h = [None, None]
        g_h[0] = pltpu.async_copy(table_hbm.at[idx_v.at[0]], rows0, g0)
        p_h.wait()

        def decode(r, carry):
            for c in range(D // 32):
                w = pack_v[r, pl.ds(c * L, L)]
                pe_v[r, pl.ds(c * 32, L)] = lax.bitcast_convert_type(
                    w << 16, jnp.float32)
                pe_v[r, pl.ds(c * 32 + L, L)] = lax.bitcast_convert_type(
                    w & jnp.int32(-65536), jnp.float32)
            return carry

        lax.fori_loop(0, POS_PER_W, decode, 0)

        for b in range(B):
            cur, nxt = b % 2, (b + 1) % 2
            if b + 1 < B:
                if o_h[nxt] is not None:
                    o_h[nxt].wait()
                g_h[nxt] = pltpu.async_copy(
                    table_hbm.at[idx_v.at[b + 1]], rows[nxt], gsem[nxt])
            g_h[cur].wait()

            def row(r, carry, cur=cur):
                for c in range(D // L):
                    sl = pl.ds(c * L, L)
                    rows[cur][r, sl] = rows[cur][r, sl] * SCALE + pe_v[r, sl]
                return carry

            lax.fori_loop(0, POS_PER_W, row, 0)
            o_h[cur] = pltpu.async_copy(
                rows[cur], out_hbm.at[pl.ds(b * S + pos0, POS_PER_W)],
                osem[cur])
        o_h[0].wait()
        o_h[1].wait()

    return emb


_emb = _make_kernel()


def kernel(x, table):
    pe = jnp.asarray(_PE_PACKED)
    out = _emb(x, table, pe)
    return out.reshape(B, S, D)
